# baseline (speedup 1.0000x reference)
import jax
import jax.numpy as jnp
from jax.experimental import pallas as pl
from jax.experimental.pallas import tpu as pltpu

TOKEN_INDEX = 0
B, S, D = 4, 2048, 1024


def _extract_body(in_ref, out_ref):
    i = pl.program_id(0)
    out_ref[i, :] = in_ref[0, TOKEN_INDEX, :]


def kernel(inputs):
    return pl.pallas_call(
        _extract_body,
        out_shape=jax.ShapeDtypeStruct((B, D), jnp.float32),
        grid=(B,),
        in_specs=[pl.BlockSpec((1, 8, D), lambda i: (i, 0, 0))],
        out_specs=pl.BlockSpec((B, D), lambda i: (0, 0)),
    )(inputs)


# speedup vs baseline: 1.6206x; 1.6206x over previous
import jax
import jax.numpy as jnp
from jax.experimental import pallas as pl
from jax.experimental.pallas import tpu as pltpu

TOKEN_INDEX = 0
B, S, D = 4, 2048, 1024


def _extract_body(in_hbm, out_ref, sem):
    copy = pltpu.make_async_copy(in_hbm.at[:, TOKEN_INDEX], out_ref, sem)
    copy.start()
    copy.wait()


def kernel(inputs):
    return pl.pallas_call(
        _extract_body,
        out_shape=jax.ShapeDtypeStruct((B, D), jnp.float32),
        in_specs=[pl.BlockSpec(memory_space=pl.ANY)],
        out_specs=pl.BlockSpec((B, D), lambda: (0, 0)),
        scratch_shapes=[pltpu.SemaphoreType.DMA],
        compiler_params=pltpu.CompilerParams(
            disable_bounds_checks=True,
            disable_semaphore_checks=True,
            skip_device_barrier=True,
        ),
    )(inputs)


# speedup vs baseline: 1.6302x; 1.0060x over previous
import jax
import jax.numpy as jnp
from jax.experimental import pallas as pl
from jax.experimental.pallas import tpu as pltpu

TOKEN_INDEX = 0
B, S, D = 4, 2048, 1024


def _extract_body(in_hbm, out_ref, sems):
    copies = [
        pltpu.make_async_copy(in_hbm.at[b, TOKEN_INDEX], out_ref.at[b], sems.at[b])
        for b in range(B)
    ]
    for c in copies:
        c.start()
    for c in copies:
        c.wait()


def kernel(inputs):
    return pl.pallas_call(
        _extract_body,
        out_shape=jax.ShapeDtypeStruct((B, D), jnp.float32),
        in_specs=[pl.BlockSpec(memory_space=pl.ANY)],
        out_specs=pl.BlockSpec((B, D), lambda: (0, 0)),
        scratch_shapes=[pltpu.SemaphoreType.DMA((B,))],
    )(inputs)
